# baseline (speedup 1.0000x reference)
import jax
import jax.numpy as jnp
from jax.experimental import pallas as pl
from jax.experimental.pallas import tpu as pltpu

_NUM_CLASSES = 10
_WEIGHT = 0.1
_B, _Y, _X = 64, 512, 512


def _loss_kernel(out_ref, tgt_ref, msk_ref, acc_ref):
    j = pl.program_id(0)

    o = out_ref[0]
    t = tgt_ref[0]
    m = msk_ref[0]

    d = o - t
    sq = d * d
    valid = m == 1
    tm = jnp.where(valid, t, float(_NUM_CLASSES))
    sqv = jnp.where(valid, sq, 0.0)

    lane = jax.lax.broadcasted_iota(jnp.int32, (2, 128), 1)
    row = jax.lax.broadcasted_iota(jnp.int32, (2, 128), 0)

    tot_s = jnp.sum(sqv)
    tot_n = jnp.sum(m).astype(jnp.float32)

    res = jnp.zeros((2, 128), jnp.float32)
    rem_s = tot_s
    rem_n = tot_n
    for c in range(_NUM_CLASSES - 1):
        eq = tm == float(c)
        s = jnp.sum(jnp.where(eq, sq, 0.0))
        n = jnp.sum(jnp.where(eq, 1.0, 0.0))
        rem_s -= s
        rem_n -= n
        is_lane = lane == c
        res = res + jnp.where(is_lane & (row == 0), s, 0.0)
        res = res + jnp.where(is_lane & (row == 1), n, 0.0)
    last = _NUM_CLASSES - 1
    res = res + jnp.where((lane == last) & (row == 0), rem_s, 0.0)
    res = res + jnp.where((lane == last) & (row == 1), rem_n, 0.0)

    @pl.when(j == 0)
    def _():
        acc_ref[...] = jnp.zeros_like(acc_ref)

    acc_ref[0] += res


def kernel(outputs, targets, mask):
    acc = pl.pallas_call(
        _loss_kernel,
        grid=(_B,),
        in_specs=[
            pl.BlockSpec((1, _Y, _X), lambda j: (j, 0, 0)),
            pl.BlockSpec((1, _Y, _X), lambda j: (j, 0, 0)),
            pl.BlockSpec((1, _Y, _X), lambda j: (j, 0, 0)),
        ],
        out_specs=pl.BlockSpec((1, 2, 128), lambda j: (0, 0, 0)),
        out_shape=jax.ShapeDtypeStruct((1, 2, 128), jnp.float32),
        compiler_params=pltpu.CompilerParams(
            dimension_semantics=("arbitrary",),
        ),
    )(outputs, targets, mask)

    tot = acc[0]
    per_class_sum = tot[0, :_NUM_CLASSES]
    class_n = tot[1, :_NUM_CLASSES]
    loss_each = jnp.where(class_n > 0, per_class_sum / jnp.maximum(class_n, 1.0), 0.0)
    loss = jnp.sum(_WEIGHT * loss_each)
    return loss, loss_each, class_n


# speedup vs baseline: 1.0960x; 1.0960x over previous
import jax
import jax.numpy as jnp
from jax.experimental import pallas as pl
from jax.experimental.pallas import tpu as pltpu

_NUM_CLASSES = 10
_WEIGHT = 0.1
_B, _Y, _X = 64, 512, 512
_BB = 2
_STEPS = _B // _BB


def _loss_kernel(out_ref, tgt_ref, msk_ref, acc_ref):
    j = pl.program_id(0)

    o = out_ref[...]
    t = tgt_ref[...]
    m = msk_ref[...]

    d = o - t
    sq = d * d
    valid = m == 1
    tm = jnp.where(valid, t, float(_NUM_CLASSES))
    sqv = jnp.where(valid, sq, 0.0)

    lane = jax.lax.broadcasted_iota(jnp.int32, (2, 128), 1)
    row = jax.lax.broadcasted_iota(jnp.int32, (2, 128), 0)

    tot_s = jnp.sum(sqv)
    tot_n = jnp.sum(m).astype(jnp.float32)

    res = jnp.zeros((2, 128), jnp.float32)
    rem_s = tot_s
    rem_n = tot_n
    for c in range(_NUM_CLASSES - 1):
        eqf = jnp.where(tm == float(c), 1.0, 0.0)
        s = jnp.sum(eqf * sq)
        n = jnp.sum(eqf)
        rem_s -= s
        rem_n -= n
        is_lane = lane == c
        res = res + jnp.where(is_lane & (row == 0), s, 0.0)
        res = res + jnp.where(is_lane & (row == 1), n, 0.0)
    last = _NUM_CLASSES - 1
    res = res + jnp.where((lane == last) & (row == 0), rem_s, 0.0)
    res = res + jnp.where((lane == last) & (row == 1), rem_n, 0.0)

    @pl.when(j == 0)
    def _():
        acc_ref[...] = jnp.zeros_like(acc_ref)

    acc_ref[0] += res


def kernel(outputs, targets, mask):
    acc = pl.pallas_call(
        _loss_kernel,
        grid=(_STEPS,),
        in_specs=[
            pl.BlockSpec((_BB, _Y, _X), lambda j: (j, 0, 0)),
            pl.BlockSpec((_BB, _Y, _X), lambda j: (j, 0, 0)),
            pl.BlockSpec((_BB, _Y, _X), lambda j: (j, 0, 0)),
        ],
        out_specs=pl.BlockSpec((1, 2, 128), lambda j: (0, 0, 0)),
        out_shape=jax.ShapeDtypeStruct((1, 2, 128), jnp.float32),
        compiler_params=pltpu.CompilerParams(
            dimension_semantics=("arbitrary",),
        ),
    )(outputs, targets, mask)

    tot = acc[0]
    per_class_sum = tot[0, :_NUM_CLASSES]
    class_n = tot[1, :_NUM_CLASSES]
    loss_each = jnp.where(class_n > 0, per_class_sum / jnp.maximum(class_n, 1.0), 0.0)
    loss = jnp.sum(_WEIGHT * loss_each)
    return loss, loss_each, class_n
